# baseline (speedup 1.0000x reference)
import functools

import jax
import jax.numpy as jnp
from jax import lax
from jax.experimental import pallas as pl
from jax.experimental.pallas import tpu as pltpu
from jax.experimental.pallas import tpu_sc as plsc

N = 10000
E = 320000
D = 128
ED = 16

NC = 2
NS = 16
NW = NC * NS
EPW = E // NW
K = 64
NBUF = 4
NPAD = 10240
NR = 4
RR = NPAD // NR
RPS = RR // NS
NCH = EPW // 16
NBMAX = (EPW + K - 1) // K + 6

BR = 2000


def _p1_body(mo_ref, ef_ref, w1m_ref, w1e_ref, b1_ref, o_ref):
    acc = jnp.dot(mo_ref[...], w1m_ref[...], preferred_element_type=jnp.float32)
    acc += jnp.dot(ef_ref[...], w1e_ref[...], preferred_element_type=jnp.float32)
    o_ref[...] = jnp.maximum(acc + b1_ref[...], 0.0)


def _phase1(mo, ef, w1m_t, w1e_t, b1):
    return pl.pallas_call(
        _p1_body,
        grid=(N // BR,),
        in_specs=[
            pl.BlockSpec((BR, D), lambda i: (i, 0)),
            pl.BlockSpec((BR, ED), lambda i: (i, 0)),
            pl.BlockSpec((D, D), lambda i: (0, 0)),
            pl.BlockSpec((ED, D), lambda i: (0, 0)),
            pl.BlockSpec((1, D), lambda i: (0, 0)),
        ],
        out_specs=pl.BlockSpec((BR, D), lambda i: (i, 0)),
        out_shape=jax.ShapeDtypeStruct((N, D), jnp.float32),
    )(mo, ef, w1m_t, w1e_t, b1)


def _sc_body(m_hbm, col_hbm, row_hbm, w_hbm, zero_hbm, out_hbm,
             col_v, row_v, w_v, ccol, crow, cw, gbufs, acc, gsems):
    c = lax.axis_index("c")
    s = lax.axis_index("s")
    wid = c * NS + s

    pltpu.sync_copy(col_hbm.at[wid], col_v)
    pltpu.sync_copy(row_hbm.at[wid], row_v)
    pltpu.sync_copy(w_hbm.at[wid], w_v)

    bufs = tuple(zip(gbufs, gsems))
    lanes = lax.iota(jnp.int32, 16)

    @pl.loop(0, NR)
    def _(r):
        lo = r * RR

        pltpu.sync_copy(zero_hbm.at[pl.ds(s * RPS, RPS)],
                        acc.at[pl.ds(s * RPS, RPS)])
        plsc.subcore_barrier()

        def chunk(t, cnt):
            sl = pl.ds(t * 16, 16)
            rv = row_v[sl]
            cv = col_v[sl]
            wv = w_v[sl]
            msk = (rv >= lo) & (rv < lo + RR)
            inc = plsc.cumsum(msk.astype(jnp.int32))
            pos = cnt + inc - 1
            pb = pos // K
            pk = pos % K
            plsc.store_scatter(crow, [pb, pk], rv - lo, mask=msk)
            plsc.store_scatter(ccol, [pb, pk], cv, mask=msk)
            plsc.store_scatter(cw, [pos], wv, mask=msk)
            return cnt + inc[15]

        cnt = lax.fori_loop(0, NCH, chunk, jnp.int32(0))

        off0 = 16 * (cnt // 16)
        keep = lanes < (cnt - off0)
        tsl = pl.ds(off0, 16)
        zi = jnp.zeros((16,), jnp.int32)
        for i in range((NBUF * K + 16) // 16 + 1):
            ppos = off0 + 16 * i + lanes
            pmask = None if i else ~keep
            plsc.store_scatter(crow, [ppos // K, ppos % K], zi, mask=pmask)
            plsc.store_scatter(ccol, [ppos // K, ppos % K], zi, mask=pmask)
        cw[tsl] = jnp.where(keep, cw[tsl], 0.0)
        for i in range(1, (NBUF * K + 16) // 16 + 1):
            cw[pl.ds(off0 + 16 * i, 16)] = jnp.zeros((16,), jnp.float32)

        plsc.subcore_barrier()

        nblk = (cnt + (K - 1)) // K

        for u, (gb, gs) in enumerate(bufs):
            @pl.when(u < nblk)
            def _():
                pltpu.async_copy(m_hbm.at[ccol.at[u]], gb, gs)

        @pl.loop(0, (nblk + (NBUF - 1)) // NBUF)
        def _(h):
            for u, (gb, gs) in enumerate(bufs):
                j = NBUF * h + u

                @pl.when(j < nblk)
                def _():
                    pltpu.make_async_copy(
                        m_hbm.at[ccol.at[j]], gb, gs).wait()
                    base = j * K

                    @pl.loop(0, K // 8)
                    def _(g):
                        for v in range(8):
                            e = g * 8 + v
                            wb = plsc.load_gather(
                                cw, [jnp.broadcast_to(base + e, (16,))])
                            for t in range(D // 16):
                                fsl = pl.ds(t * 16, 16)
                                gb[e, fsl] = gb[e, fsl] * wb
                    pltpu.sync_copy(gb, acc.at[crow.at[j]], add=True)

                    @pl.when(j + NBUF < nblk)
                    def _():
                        pltpu.async_copy(m_hbm.at[ccol.at[j + NBUF]], gb, gs)

        plsc.subcore_barrier()
        pltpu.sync_copy(acc.at[pl.ds(s * RPS, RPS)],
                        out_hbm.at[c, r, pl.ds(s * RPS, RPS)])
        plsc.subcore_barrier()


def _phase2(m, col, row, w, zeros):
    mesh = plsc.VectorSubcoreMesh(core_axis_name="c", subcore_axis_name="s")
    f = pl.kernel(
        _sc_body,
        out_type=jax.ShapeDtypeStruct((NC, NR, RR, D), jnp.float32),
        mesh=mesh,
        scratch_types=[
            pltpu.VMEM((EPW,), jnp.int32),
            pltpu.VMEM((EPW,), jnp.int32),
            pltpu.VMEM((EPW,), jnp.float32),
            pltpu.VMEM((NBMAX, K), jnp.int32),
            pltpu.VMEM((NBMAX, K), jnp.int32),
            pltpu.VMEM((EPW + NBUF * K + 64,), jnp.float32),
            [pltpu.VMEM((K, D), jnp.float32)] * NBUF,
            pltpu.VMEM_SHARED((RR, D), jnp.float32),
            [pltpu.SemaphoreType.DMA] * NBUF,
        ],
        compiler_params=pltpu.CompilerParams(
            needs_layout_passes=False, use_tc_tiling_on_sc=False),
    )
    return f(m, col, row, w, zeros)


def _p3_body(p0_ref, p1_ref, mo_ref, w2_ref, b2_ref, wih_ref, whh_ref,
             bih_ref, bhh_ref, o_ref):
    agg = p0_ref[...] + p1_ref[...]
    m2 = jnp.maximum(
        jnp.dot(agg, w2_ref[...], preferred_element_type=jnp.float32)
        + b2_ref[...], 0.0)
    gi = jnp.dot(m2, wih_ref[...], preferred_element_type=jnp.float32) + bih_ref[...]
    mo = mo_ref[...]
    gh = jnp.dot(mo, whh_ref[...], preferred_element_type=jnp.float32) + bhh_ref[...]
    r = jax.nn.sigmoid(gi[:, :D] + gh[:, :D])
    z = jax.nn.sigmoid(gi[:, D:2 * D] + gh[:, D:2 * D])
    n = jnp.tanh(gi[:, 2 * D:] + r * gh[:, 2 * D:])
    o_ref[...] = (1.0 - z) * n + z * mo


def _phase3(p0, p1, mo, w2_t, b2, wih_t, whh_t, bih, bhh):
    return pl.pallas_call(
        _p3_body,
        grid=(N // BR,),
        in_specs=[
            pl.BlockSpec((BR, D), lambda i: (i, 0)),
            pl.BlockSpec((BR, D), lambda i: (i, 0)),
            pl.BlockSpec((BR, D), lambda i: (i, 0)),
            pl.BlockSpec((D, D), lambda i: (0, 0)),
            pl.BlockSpec((1, D), lambda i: (0, 0)),
            pl.BlockSpec((D, 3 * D), lambda i: (0, 0)),
            pl.BlockSpec((D, 3 * D), lambda i: (0, 0)),
            pl.BlockSpec((1, 3 * D), lambda i: (0, 0)),
            pl.BlockSpec((1, 3 * D), lambda i: (0, 0)),
        ],
        out_specs=pl.BlockSpec((BR, D), lambda i: (i, 0)),
        out_shape=jax.ShapeDtypeStruct((N, D), jnp.float32),
    )(p0, p1, mo, w2_t, b2, wih_t, whh_t, bih, bhh)


def kernel(node_feat, node_aux, edge_feat, message_old, edge_index, edge_weight,
           W1, b1, W2, b2, W_ih, W_hh, b_ih, b_hh):
    del node_feat, node_aux
    row = edge_index[0].reshape(NW, EPW)
    col = edge_index[1].reshape(NW, EPW)
    w = edge_weight.reshape(NW, EPW)
    w1m_t = W1[:, :D].T
    w1e_t = W1[:, D:].T
    b1r = b1.reshape(1, D)
    w2_t = W2.T
    b2r = b2.reshape(1, D)
    wih_t = W_ih.T
    whh_t = W_hh.T
    bihr = b_ih.reshape(1, 3 * D)
    bhhr = b_hh.reshape(1, 3 * D)
    zeros = jnp.zeros((RR, D), jnp.float32)

    m = _phase1(message_old, edge_feat, w1m_t, w1e_t, b1r)
    parts = _phase2(m, col, row, w, zeros)
    p0 = parts[0].reshape(NPAD, D)[:N]
    p1 = parts[1].reshape(NPAD, D)[:N]
    return _phase3(p0, p1, message_old, w2_t, b2r, wih_t, whh_t, bihr, bhhr)


# speedup vs baseline: 1.0053x; 1.0053x over previous
import functools

import jax
import jax.numpy as jnp
from jax import lax
from jax.experimental import pallas as pl
from jax.experimental.pallas import tpu as pltpu
from jax.experimental.pallas import tpu_sc as plsc

N = 10000
E = 320000
D = 128
ED = 16

NC = 2
NS = 16
NW = NC * NS
EPW = E // NW
K = 64
NBUF = 6
NPAD = 10240
NR = 4
RR = NPAD // NR
RPS = RR // NS
NCH = EPW // 16
NBMAX = (EPW + K - 1) // K + 6

BR = 2000


def _p1_body(mo_ref, ef_ref, w1m_ref, w1e_ref, b1_ref, o_ref):
    acc = jnp.dot(mo_ref[...], w1m_ref[...], preferred_element_type=jnp.float32)
    acc += jnp.dot(ef_ref[...], w1e_ref[...], preferred_element_type=jnp.float32)
    o_ref[...] = jnp.maximum(acc + b1_ref[...], 0.0)


def _phase1(mo, ef, w1m_t, w1e_t, b1):
    return pl.pallas_call(
        _p1_body,
        grid=(N // BR,),
        in_specs=[
            pl.BlockSpec((BR, D), lambda i: (i, 0)),
            pl.BlockSpec((BR, ED), lambda i: (i, 0)),
            pl.BlockSpec((D, D), lambda i: (0, 0)),
            pl.BlockSpec((ED, D), lambda i: (0, 0)),
            pl.BlockSpec((1, D), lambda i: (0, 0)),
        ],
        out_specs=pl.BlockSpec((BR, D), lambda i: (i, 0)),
        out_shape=jax.ShapeDtypeStruct((N, D), jnp.float32),
    )(mo, ef, w1m_t, w1e_t, b1)


def _sc_body(m_hbm, col_hbm, row_hbm, w_hbm, zero_hbm, out_hbm,
             col_v, row_v, w_v, ccol, crow, cw, gbufs, acc, gsems):
    c = lax.axis_index("c")
    s = lax.axis_index("s")
    wid = c * NS + s

    pltpu.sync_copy(col_hbm.at[wid], col_v)
    pltpu.sync_copy(row_hbm.at[wid], row_v)
    pltpu.sync_copy(w_hbm.at[wid], w_v)

    bufs = tuple(zip(gbufs, gsems))
    lanes = lax.iota(jnp.int32, 16)

    @pl.loop(0, NR)
    def _(r):
        lo = r * RR

        pltpu.sync_copy(zero_hbm.at[pl.ds(s * RPS, RPS)],
                        acc.at[pl.ds(s * RPS, RPS)])
        plsc.subcore_barrier()

        def chunk(t, cnt):
            sl = pl.ds(t * 16, 16)
            rv = row_v[sl]
            cv = col_v[sl]
            wv = w_v[sl]
            msk = (rv >= lo) & (rv < lo + RR)
            inc = plsc.cumsum(msk.astype(jnp.int32))
            pos = cnt + inc - 1
            pb = pos // K
            pk = pos % K
            plsc.store_scatter(crow, [pb, pk], rv - lo, mask=msk)
            plsc.store_scatter(ccol, [pb, pk], cv, mask=msk)
            plsc.store_scatter(cw, [pos], wv, mask=msk)
            return cnt + inc[15]

        cnt = lax.fori_loop(0, NCH, chunk, jnp.int32(0))

        off0 = 16 * (cnt // 16)
        keep = lanes < (cnt - off0)
        tsl = pl.ds(off0, 16)
        zi = jnp.zeros((16,), jnp.int32)
        for i in range((NBUF * K + 16) // 16 + 1):
            ppos = off0 + 16 * i + lanes
            pmask = None if i else ~keep
            plsc.store_scatter(crow, [ppos // K, ppos % K], zi, mask=pmask)
            plsc.store_scatter(ccol, [ppos // K, ppos % K], zi, mask=pmask)
        cw[tsl] = jnp.where(keep, cw[tsl], 0.0)
        for i in range(1, (NBUF * K + 16) // 16 + 1):
            cw[pl.ds(off0 + 16 * i, 16)] = jnp.zeros((16,), jnp.float32)

        plsc.subcore_barrier()

        nblk = (cnt + (K - 1)) // K

        for u, (gb, gs) in enumerate(bufs):
            @pl.when(u < nblk)
            def _():
                pltpu.async_copy(m_hbm.at[ccol.at[u]], gb, gs)

        @pl.loop(0, (nblk + (NBUF - 1)) // NBUF)
        def _(h):
            for u, (gb, gs) in enumerate(bufs):
                j = NBUF * h + u

                @pl.when(j < nblk)
                def _():
                    pltpu.make_async_copy(
                        m_hbm.at[ccol.at[j]], gb, gs).wait()
                    base = j * K

                    @pl.loop(0, K // 8)
                    def _(g):
                        for v in range(8):
                            e = g * 8 + v
                            wb = plsc.load_gather(
                                cw, [jnp.broadcast_to(base + e, (16,))])
                            for t in range(D // 16):
                                fsl = pl.ds(t * 16, 16)
                                gb[e, fsl] = gb[e, fsl] * wb
                    pltpu.sync_copy(gb, acc.at[crow.at[j]], add=True)

                    @pl.when(j + NBUF < nblk)
                    def _():
                        pltpu.async_copy(m_hbm.at[ccol.at[j + NBUF]], gb, gs)

        plsc.subcore_barrier()
        pltpu.sync_copy(acc.at[pl.ds(s * RPS, RPS)],
                        out_hbm.at[c, r, pl.ds(s * RPS, RPS)])
        plsc.subcore_barrier()


def _phase2(m, col, row, w, zeros):
    mesh = plsc.VectorSubcoreMesh(core_axis_name="c", subcore_axis_name="s")
    f = pl.kernel(
        _sc_body,
        out_type=jax.ShapeDtypeStruct((NC, NR, RR, D), jnp.float32),
        mesh=mesh,
        scratch_types=[
            pltpu.VMEM((EPW,), jnp.int32),
            pltpu.VMEM((EPW,), jnp.int32),
            pltpu.VMEM((EPW,), jnp.float32),
            pltpu.VMEM((NBMAX, K), jnp.int32),
            pltpu.VMEM((NBMAX, K), jnp.int32),
            pltpu.VMEM((EPW + NBUF * K + 64,), jnp.float32),
            [pltpu.VMEM((K, D), jnp.float32)] * NBUF,
            pltpu.VMEM_SHARED((RR, D), jnp.float32),
            [pltpu.SemaphoreType.DMA] * NBUF,
        ],
        compiler_params=pltpu.CompilerParams(
            needs_layout_passes=False, use_tc_tiling_on_sc=False),
    )
    return f(m, col, row, w, zeros)


def _p3_body(p0_ref, p1_ref, mo_ref, w2_ref, b2_ref, wih_ref, whh_ref,
             bih_ref, bhh_ref, o_ref):
    agg = p0_ref[...] + p1_ref[...]
    m2 = jnp.maximum(
        jnp.dot(agg, w2_ref[...], preferred_element_type=jnp.float32)
        + b2_ref[...], 0.0)
    gi = jnp.dot(m2, wih_ref[...], preferred_element_type=jnp.float32) + bih_ref[...]
    mo = mo_ref[...]
    gh = jnp.dot(mo, whh_ref[...], preferred_element_type=jnp.float32) + bhh_ref[...]
    r = jax.nn.sigmoid(gi[:, :D] + gh[:, :D])
    z = jax.nn.sigmoid(gi[:, D:2 * D] + gh[:, D:2 * D])
    n = jnp.tanh(gi[:, 2 * D:] + r * gh[:, 2 * D:])
    o_ref[...] = (1.0 - z) * n + z * mo


def _phase3(p0, p1, mo, w2_t, b2, wih_t, whh_t, bih, bhh):
    return pl.pallas_call(
        _p3_body,
        grid=(N // BR,),
        in_specs=[
            pl.BlockSpec((BR, D), lambda i: (i, 0)),
            pl.BlockSpec((BR, D), lambda i: (i, 0)),
            pl.BlockSpec((BR, D), lambda i: (i, 0)),
            pl.BlockSpec((D, D), lambda i: (0, 0)),
            pl.BlockSpec((1, D), lambda i: (0, 0)),
            pl.BlockSpec((D, 3 * D), lambda i: (0, 0)),
            pl.BlockSpec((D, 3 * D), lambda i: (0, 0)),
            pl.BlockSpec((1, 3 * D), lambda i: (0, 0)),
            pl.BlockSpec((1, 3 * D), lambda i: (0, 0)),
        ],
        out_specs=pl.BlockSpec((BR, D), lambda i: (i, 0)),
        out_shape=jax.ShapeDtypeStruct((N, D), jnp.float32),
    )(p0, p1, mo, w2_t, b2, wih_t, whh_t, bih, bhh)


def kernel(node_feat, node_aux, edge_feat, message_old, edge_index, edge_weight,
           W1, b1, W2, b2, W_ih, W_hh, b_ih, b_hh):
    del node_feat, node_aux
    row = edge_index[0].reshape(NW, EPW)
    col = edge_index[1].reshape(NW, EPW)
    w = edge_weight.reshape(NW, EPW)
    w1m_t = W1[:, :D].T
    w1e_t = W1[:, D:].T
    b1r = b1.reshape(1, D)
    w2_t = W2.T
    b2r = b2.reshape(1, D)
    wih_t = W_ih.T
    whh_t = W_hh.T
    bihr = b_ih.reshape(1, 3 * D)
    bhhr = b_hh.reshape(1, 3 * D)
    zeros = jnp.zeros((RR, D), jnp.float32)

    m = _phase1(message_old, edge_feat, w1m_t, w1e_t, b1r)
    parts = _phase2(m, col, row, w, zeros)
    p0 = parts[0].reshape(NPAD, D)[:N]
    p1 = parts[1].reshape(NPAD, D)[:N]
    return _phase3(p0, p1, message_old, w2_t, b2r, wih_t, whh_t, bihr, bhhr)


# speedup vs baseline: 1.1005x; 1.0947x over previous
import functools

import jax
import jax.numpy as jnp
from jax import lax
from jax.experimental import pallas as pl
from jax.experimental.pallas import tpu as pltpu
from jax.experimental.pallas import tpu_sc as plsc

N = 10000
E = 320000
D = 128
ED = 16

NC = 2
NS = 16
NW = NC * NS
EPW = E // NW
K = 64
NBUF = 6
PD = NBUF - 2
NPAD = 10240
NR = 4
RR = NPAD // NR
RPS = RR // NS
NCH = EPW // 16
NBMAX = (EPW + K - 1) // K + 6

BR = 2000


def _p1_body(mo_ref, ef_ref, w1m_ref, w1e_ref, b1_ref, o_ref):
    acc = jnp.dot(mo_ref[...], w1m_ref[...], preferred_element_type=jnp.float32)
    acc += jnp.dot(ef_ref[...], w1e_ref[...], preferred_element_type=jnp.float32)
    o_ref[...] = jnp.maximum(acc + b1_ref[...], 0.0)


def _phase1(mo, ef, w1m_t, w1e_t, b1):
    return pl.pallas_call(
        _p1_body,
        grid=(N // BR,),
        in_specs=[
            pl.BlockSpec((BR, D), lambda i: (i, 0)),
            pl.BlockSpec((BR, ED), lambda i: (i, 0)),
            pl.BlockSpec((D, D), lambda i: (0, 0)),
            pl.BlockSpec((ED, D), lambda i: (0, 0)),
            pl.BlockSpec((1, D), lambda i: (0, 0)),
        ],
        out_specs=pl.BlockSpec((BR, D), lambda i: (i, 0)),
        out_shape=jax.ShapeDtypeStruct((N, D), jnp.float32),
    )(mo, ef, w1m_t, w1e_t, b1)


def _sc_body(m_hbm, col_hbm, row_hbm, w_hbm, zero_hbm, out_hbm,
             col_v, row_v, w_v, ccol, crow, cw, gbufs, acc, gsems, ssems):
    c = lax.axis_index("c")
    s = lax.axis_index("s")
    wid = c * NS + s

    pltpu.sync_copy(col_hbm.at[wid], col_v)
    pltpu.sync_copy(row_hbm.at[wid], row_v)
    pltpu.sync_copy(w_hbm.at[wid], w_v)

    bufs = tuple(zip(gbufs, gsems))
    lanes = lax.iota(jnp.int32, 16)

    @pl.loop(0, NR)
    def _(r):
        lo = r * RR

        pltpu.sync_copy(zero_hbm.at[pl.ds(s * RPS, RPS)],
                        acc.at[pl.ds(s * RPS, RPS)])
        plsc.subcore_barrier()

        def chunk(t, cnt):
            sl = pl.ds(t * 16, 16)
            rv = row_v[sl]
            cv = col_v[sl]
            wv = w_v[sl]
            msk = (rv >= lo) & (rv < lo + RR)
            inc = plsc.cumsum(msk.astype(jnp.int32))
            pos = cnt + inc - 1
            pb = pos // K
            pk = pos % K
            plsc.store_scatter(crow, [pb, pk], rv - lo, mask=msk)
            plsc.store_scatter(ccol, [pb, pk], cv, mask=msk)
            plsc.store_scatter(cw, [pos], wv, mask=msk)
            return cnt + inc[15]

        cnt = lax.fori_loop(0, NCH, chunk, jnp.int32(0))

        off0 = 16 * (cnt // 16)
        keep = lanes < (cnt - off0)
        tsl = pl.ds(off0, 16)
        zi = jnp.zeros((16,), jnp.int32)
        for i in range((NBUF * K + 16) // 16 + 1):
            ppos = off0 + 16 * i + lanes
            pmask = None if i else ~keep
            plsc.store_scatter(crow, [ppos // K, ppos % K], zi, mask=pmask)
            plsc.store_scatter(ccol, [ppos // K, ppos % K], zi, mask=pmask)
        cw[tsl] = jnp.where(keep, cw[tsl], 0.0)
        for i in range(1, (NBUF * K + 16) // 16 + 1):
            cw[pl.ds(off0 + 16 * i, 16)] = jnp.zeros((16,), jnp.float32)

        plsc.subcore_barrier()

        nblk = (cnt + (K - 1)) // K

        for u in range(PD):
            @pl.when(u < nblk)
            def _():
                pltpu.async_copy(m_hbm.at[ccol.at[u]], bufs[u][0], bufs[u][1])

        @pl.loop(0, (nblk + (NBUF - 1)) // NBUF)
        def _(h):
            for u, (gb, gs) in enumerate(bufs):
                j = NBUF * h + u
                un = (u + PD) % NBUF
                gbn, gsn = bufs[un]

                @pl.when(j < nblk)
                def _():
                    pltpu.make_async_copy(
                        m_hbm.at[ccol.at[j]], gb, gs).wait()
                    base = j * K

                    @pl.loop(0, K // 8)
                    def _(g):
                        for v in range(8):
                            e = g * 8 + v
                            wb = plsc.load_gather(
                                cw, [jnp.broadcast_to(base + e, (16,))])
                            for t in range(D // 16):
                                fsl = pl.ds(t * 16, 16)
                                gb[e, fsl] = gb[e, fsl] * wb
                    pltpu.async_copy(gb, acc.at[crow.at[j]], ssems[u],
                                     add=True)

                    jn = j + PD

                    @pl.when(jn < nblk)
                    def _():
                        @pl.when(jn - NBUF >= 0)
                        def _():
                            pltpu.make_async_copy(
                                gbn, acc.at[crow.at[jn - NBUF]],
                                ssems[un]).wait()

                        pltpu.async_copy(m_hbm.at[ccol.at[jn]], gbn, gsn)

        for u, (gb, gs) in enumerate(bufs):
            @pl.when(u < nblk)
            def _():
                bu = ((nblk - 1 - u) // NBUF) * NBUF + u
                pltpu.make_async_copy(
                    gb, acc.at[crow.at[bu]], ssems[u]).wait()

        plsc.subcore_barrier()
        pltpu.sync_copy(acc.at[pl.ds(s * RPS, RPS)],
                        out_hbm.at[c, r, pl.ds(s * RPS, RPS)])
        plsc.subcore_barrier()


def _phase2(m, col, row, w, zeros):
    mesh = plsc.VectorSubcoreMesh(core_axis_name="c", subcore_axis_name="s")
    f = pl.kernel(
        _sc_body,
        out_type=jax.ShapeDtypeStruct((NC, NR, RR, D), jnp.float32),
        mesh=mesh,
        scratch_types=[
            pltpu.VMEM((EPW,), jnp.int32),
            pltpu.VMEM((EPW,), jnp.int32),
            pltpu.VMEM((EPW,), jnp.float32),
            pltpu.VMEM((NBMAX, K), jnp.int32),
            pltpu.VMEM((NBMAX, K), jnp.int32),
            pltpu.VMEM((EPW + NBUF * K + 64,), jnp.float32),
            [pltpu.VMEM((K, D), jnp.float32)] * NBUF,
            pltpu.VMEM_SHARED((RR, D), jnp.float32),
            [pltpu.SemaphoreType.DMA] * NBUF,
            [pltpu.SemaphoreType.DMA] * NBUF,
        ],
        compiler_params=pltpu.CompilerParams(
            needs_layout_passes=False, use_tc_tiling_on_sc=False),
    )
    return f(m, col, row, w, zeros)


def _p3_body(p0_ref, p1_ref, mo_ref, w2_ref, b2_ref, wih_ref, whh_ref,
             bih_ref, bhh_ref, o_ref):
    agg = p0_ref[...] + p1_ref[...]
    m2 = jnp.maximum(
        jnp.dot(agg, w2_ref[...], preferred_element_type=jnp.float32)
        + b2_ref[...], 0.0)
    gi = jnp.dot(m2, wih_ref[...], preferred_element_type=jnp.float32) + bih_ref[...]
    mo = mo_ref[...]
    gh = jnp.dot(mo, whh_ref[...], preferred_element_type=jnp.float32) + bhh_ref[...]
    r = jax.nn.sigmoid(gi[:, :D] + gh[:, :D])
    z = jax.nn.sigmoid(gi[:, D:2 * D] + gh[:, D:2 * D])
    n = jnp.tanh(gi[:, 2 * D:] + r * gh[:, 2 * D:])
    o_ref[...] = (1.0 - z) * n + z * mo


def _phase3(p0, p1, mo, w2_t, b2, wih_t, whh_t, bih, bhh):
    return pl.pallas_call(
        _p3_body,
        grid=(N // BR,),
        in_specs=[
            pl.BlockSpec((BR, D), lambda i: (i, 0)),
            pl.BlockSpec((BR, D), lambda i: (i, 0)),
            pl.BlockSpec((BR, D), lambda i: (i, 0)),
            pl.BlockSpec((D, D), lambda i: (0, 0)),
            pl.BlockSpec((1, D), lambda i: (0, 0)),
            pl.BlockSpec((D, 3 * D), lambda i: (0, 0)),
            pl.BlockSpec((D, 3 * D), lambda i: (0, 0)),
            pl.BlockSpec((1, 3 * D), lambda i: (0, 0)),
            pl.BlockSpec((1, 3 * D), lambda i: (0, 0)),
        ],
        out_specs=pl.BlockSpec((BR, D), lambda i: (i, 0)),
        out_shape=jax.ShapeDtypeStruct((N, D), jnp.float32),
    )(p0, p1, mo, w2_t, b2, wih_t, whh_t, bih, bhh)


def kernel(node_feat, node_aux, edge_feat, message_old, edge_index, edge_weight,
           W1, b1, W2, b2, W_ih, W_hh, b_ih, b_hh):
    del node_feat, node_aux
    row = edge_index[0].reshape(NW, EPW)
    col = edge_index[1].reshape(NW, EPW)
    w = edge_weight.reshape(NW, EPW)
    w1m_t = W1[:, :D].T
    w1e_t = W1[:, D:].T
    b1r = b1.reshape(1, D)
    w2_t = W2.T
    b2r = b2.reshape(1, D)
    wih_t = W_ih.T
    whh_t = W_hh.T
    bihr = b_ih.reshape(1, 3 * D)
    bhhr = b_hh.reshape(1, 3 * D)
    zeros = jnp.zeros((RR, D), jnp.float32)

    m = _phase1(message_old, edge_feat, w1m_t, w1e_t, b1r)
    parts = _phase2(m, col, row, w, zeros)
    p0 = parts[0].reshape(NPAD, D)[:N]
    p1 = parts[1].reshape(NPAD, D)[:N]
    return _phase3(p0, p1, message_old, w2_t, b2r, wih_t, whh_t, bihr, bhhr)
